# baseline (speedup 1.0000x reference)
import functools

import jax
import jax.numpy as jnp
from jax import lax
from jax.experimental import pallas as pl
from jax.experimental.pallas import tpu as pltpu
from jax.experimental.pallas import tpu_sc as plsc

_NUM_CORES = 2
_NUM_SUBCORES = 16
_NW = _NUM_CORES * _NUM_SUBCORES

_NBUF = 6


@functools.lru_cache(maxsize=None)
def _make_gather(num_rows: int, dim: int, bsz: int, fields: int):
    assert bsz % _NW == 0
    bpw = bsz // _NW

    mesh = plsc.VectorSubcoreMesh(core_axis_name="c", subcore_axis_name="s")

    @functools.partial(
        pl.kernel,
        out_type=jax.ShapeDtypeStruct((fields, bsz, dim), jnp.float32),
        mesh=mesh,
        compiler_params=pltpu.CompilerParams(use_tc_tiling_on_sc=True),
        scratch_types=[
            pltpu.VMEM((fields * bpw,), jnp.int32),
            [pltpu.VMEM((bpw // 2, dim), jnp.float32) for _ in range(_NBUF)],
            [pltpu.SemaphoreType.DMA for _ in range(_NBUF)],
            [pltpu.SemaphoreType.DMA for _ in range(_NBUF)],
            pltpu.SemaphoreType.DMA,
        ],
    )
    def gather(idx_hbm, tbl_hbm, out_hbm, idx_v, bufs, gsems, wsems, isem):
        wid = lax.axis_index("s") * _NUM_CORES + lax.axis_index("c")
        b0 = wid * bpw
        hb = bpw // 2
        nchunk = 2 * fields

        idescs = []
        for f in range(fields):
            idescs.append(pltpu.async_copy(
                idx_hbm.at[pl.ds(f * bsz + b0, bpw)],
                idx_v.at[pl.ds(f * bpw, bpw)], isem))
        for d in idescs:
            d.wait()

        def start_gather(g):
            b = g % _NBUF
            return pltpu.async_copy(
                tbl_hbm.at[idx_v.at[pl.ds(g * hb, hb)]], bufs[b], gsems[b])

        def write_out(g):
            b = g % _NBUF
            f, h = g // 2, g % 2
            return pltpu.async_copy(
                bufs[b], out_hbm.at[f, pl.ds(b0 + h * hb, hb)], wsems[b])

        gd = [None] * nchunk
        wd = [None] * nchunk
        for g in range(nchunk):
            if g >= _NBUF:
                wd[g - _NBUF].wait()
            gd[g] = start_gather(g)
            if g >= 1:
                gd[g - 1].wait()
                wd[g - 1] = write_out(g - 1)
        gd[nchunk - 1].wait()
        wd[nchunk - 1] = write_out(nchunk - 1)
        for g in range(max(0, nchunk - _NBUF + 1), nchunk):
            wd[g].wait()

    return gather


def _mm_body(a_ref, b_ref, o_ref):
    o_ref[...] = jnp.dot(a_ref[...], b_ref[...],
                         preferred_element_type=jnp.float32)


@functools.lru_cache(maxsize=None)
def _make_mm(m: int, k: int, n: int):
    return pl.pallas_call(
        _mm_body,
        out_shape=jax.ShapeDtypeStruct((m, n), jnp.float32),
    )


@jax.jit
def kernel(input_batch_inputs_, weight, mat1, mat2):
    bsz, fields = input_batch_inputs_.shape
    num_rows, dim = weight.shape
    idx_t = jnp.swapaxes(input_batch_inputs_, 0, 1).reshape(-1)
    emb_t = _make_gather(num_rows, dim, bsz, fields)(idx_t, weight)
    emb = jnp.transpose(emb_t, (1, 0, 2))
    mm = _make_mm(mat1.shape[0], mat1.shape[1], mat2.shape[1])(mat1, mat2)
    return emb, mm
